# Initial kernel scaffold; baseline (speedup 1.0000x reference)
import jax
import jax.numpy as jnp
from jax.experimental import pallas as pl


def kernel(x, W1, b1, W2, b2):
    raise NotImplementedError("write your pallas kernel here")


# speedup vs baseline: 2.2470x; 2.2470x over previous
import numpy as np
import jax
import jax.numpy as jnp
from jax import lax
from jax.experimental import pallas as pl
from jax.experimental.pallas import tpu as pltpu
from jax.experimental.pallas import tpu_sc as plsc

BATCH = 1024
NPART = 16384
NQ = 20
L = 16
NC, NS = 2, 16
NW = NC * NS
ROWS_PER_TILE = BATCH // NW
NVEC = NPART // L
NRP = 48
DUMMY = 47
QPAD = 32
SIGN = np.int32(-2147483648)
MASK31 = np.int32(2147483647)


def _rank_consts():
    qs = np.array([
        1028443341, 1036478745, 1041511909, 1044690750, 1047869591,
        1049812216, 1051401637, 1052991057, 1054580478, 1056169898,
        1057361963, 1058156674, 1058951384, 1059746094, 1060540804,
        1061335514, 1062130225, 1062924935, 1063719645, 1064514355,
    ], dtype=np.uint32).view(np.float32)
    pos = (qs * np.float32(NPART - 1)).astype(np.float32)
    lo = np.clip(np.floor(pos), 0, NPART - 1).astype(np.int64)
    hi = np.clip(np.ceil(pos), 0, NPART - 1).astype(np.int64)
    w = (pos - lo.astype(np.float32)).astype(np.float32)
    ranks = np.empty(NRP, dtype=np.int32)
    ranks[0:2 * NQ:2] = lo
    ranks[1:2 * NQ:2] = hi
    ranks[2 * NQ:] = hi[-1]
    return ranks, w


_RANKS, _HIW = _rank_consts()


def _to_sortable(xf):
    bits = lax.bitcast_convert_type(xf, jnp.int32)
    m = lax.shift_right_arithmetic(bits, 31)
    return lax.bitwise_xor(bits, lax.bitwise_or(SIGN, lax.bitwise_and(MASK31, m)))


def _from_sortable(u):
    m = lax.shift_right_arithmetic(u, 31)
    notm = lax.bitwise_xor(m, np.int32(-1))
    bits = lax.bitwise_xor(u, lax.bitwise_or(SIGN, lax.bitwise_and(MASK31, notm)))
    return lax.bitcast_convert_type(bits, jnp.float32)


def _srl(x, n):
    return lax.shift_right_logical(x, np.int32(n))


def _sc_quantile_body(x_hbm, rk_hbm, hw_hbm, qout_hbm, data0, hist1, table2,
                      table3, table4, hist2, stage, vstage, qrow, rk_v, hw_v):
    wid = lax.axis_index("s") * NC + lax.axis_index("c")
    lane = lax.iota(jnp.int32, L)
    zeros = lane * 0
    ones = zeros + 1
    dummy = zeros + DUMMY
    fzeros = zeros.astype(jnp.float32)
    pltpu.sync_copy(rk_hbm, rk_v)
    pltpu.sync_copy(hw_hbm, hw_v)
    kvecs = [rk_v[pl.ds(i * L, L)] for i in range(3)]
    hiw0 = hw_v[pl.ds(0, L)]
    hiw1 = hw_v[pl.ds(L, L)]

    def init1(i, _):
        hist1[pl.ds(i * L, L)] = zeros
        table2[pl.ds(i * L, L)] = dummy
        return 0
    lax.fori_loop(0, 2048 // L, init1, 0)

    def init2(i, _):
        hist2[pl.ds(i * L, L)] = zeros
        table3[pl.ds(i * L, L)] = dummy
        table4[pl.ds(i * L, L)] = dummy
        return 0
    lax.fori_loop(0, (NRP * 128) // L, init2, 0)

    def load_u(v):
        return _to_sortable(data0[pl.ds(v * L, L)])

    def refine_pass(level, shift, table, pkeys, kks):
        stage[pl.ds(1, L)] = pkeys[0]
        stage[pl.ds(1 + L, L)] = pkeys[1]
        stage[pl.ds(1 + 2 * L, L)] = pkeys[2]
        sh = [stage[pl.ds(i * L, L)] for i in range(3)]
        f0 = jnp.logical_or(pkeys[0] != sh[0], lane == 0)
        f1 = pkeys[1] != sh[1]
        f2 = pkeys[2] != sh[2]
        fi = [f0.astype(jnp.int32), f1.astype(jnp.int32), f2.astype(jnp.int32)]
        n0 = jnp.sum(fi[0])
        n1 = jnp.sum(fi[1])
        slots = [plsc.cumsum(fi[0]) - 1,
                 plsc.cumsum(fi[1]) + (n0 - 1),
                 plsc.cumsum(fi[2]) + (n0 + n1 - 1)]
        plsc.store_scatter(table, [pkeys[0]], slots[0], mask=f0)
        plsc.store_scatter(table, [pkeys[1]], slots[1], mask=f1)
        plsc.store_scatter(table, [pkeys[2]], slots[2], mask=f2)

        def dp(v, _):
            u = load_u(v)
            s = plsc.load_gather(table2, [_srl(u, 21)])
            if level >= 3:
                k2 = lax.bitwise_and(_srl(u, 14), np.int32(127))
                s = plsc.load_gather(table3, [s * 128 + k2])
            if level >= 4:
                k3 = lax.bitwise_and(_srl(u, 7), np.int32(127))
                s = plsc.load_gather(table4, [s * 128 + k3])
            key = lax.bitwise_and(_srl(u, shift), np.int32(127))
            plsc.addupdate_scatter(hist2, [key * NRP + s], ones)
            return 0
        lax.fori_loop(0, NVEC, dp, 0)

        def sc(b, carry):
            acc, bins, bef = carry
            base = b * NRP
            acc2, bins2, bef2 = [], [], []
            for g in range(3):
                h = plsc.load_gather(hist2, [slots[g] + base])
                a = acc[g] + h
                m = a <= kks[g]
                acc2.append(a)
                bins2.append(bins[g] + m.astype(jnp.int32))
                bef2.append(jnp.where(m, a, bef[g]))
            hist2[pl.ds(base, L)] = zeros
            hist2[pl.ds(base + L, L)] = zeros
            hist2[pl.ds(base + 2 * L, L)] = zeros
            return (tuple(acc2), tuple(bins2), tuple(bef2))
        z3 = (zeros, zeros, zeros)
        _, bins, bef = lax.fori_loop(0, 128, sc, (z3, z3, z3))
        new_kks = tuple(kks[g] - bef[g] for g in range(3))
        return bins, new_kks, slots

    def row_body(i, _):
        row = wid * ROWS_PER_TILE + i
        pltpu.sync_copy(x_hbm.at[row], data0)

        def p1(v, _):
            u = load_u(v)
            plsc.addupdate_scatter(hist1, [_srl(u, 21)], ones)
            return 0
        lax.fori_loop(0, NVEC, p1, 0)

        def cs(b, carry):
            h = hist1[pl.ds(b * L, L)]
            hist1[pl.ds(b * L, L)] = plsc.cumsum(h) + carry
            return carry + jnp.sum(h)
        lax.fori_loop(0, 2048 // L, cs, np.int32(0))

        def bsearch(kvec):
            b = zeros
            for j in (1024, 512, 256, 128, 64, 32, 16, 8, 4, 2, 1):
                val = plsc.load_gather(hist1, [b + (j - 1)])
                b = jnp.where(val <= kvec, b + j, b)
            return b
        b1 = [bsearch(kvecs[g]) for g in range(3)]
        kks = []
        for g in range(3):
            prev = plsc.load_gather(hist1, [jnp.maximum(b1[g] - 1, 0)])
            bef = jnp.where(b1[g] > 0, prev, 0)
            kks.append(kvecs[g] - bef)
        kks = tuple(kks)

        def clr(b, _):
            hist1[pl.ds(b * L, L)] = zeros
            return 0
        lax.fori_loop(0, 2048 // L, clr, 0)

        pk2 = tuple(b1)
        b2, kks, s2 = refine_pass(2, 14, table2, pk2, kks)
        pk3 = tuple(s2[g] * 128 + b2[g] for g in range(3))
        b3, kks, s3 = refine_pass(3, 7, table3, pk3, kks)
        pk4 = tuple(s3[g] * 128 + b3[g] for g in range(3))
        b4, kks, s4 = refine_pass(4, 0, table4, pk4, kks)

        for g in range(3):
            plsc.store_scatter(table2, [pk2[g]], dummy)
            plsc.store_scatter(table3, [pk3[g]], dummy)
            plsc.store_scatter(table4, [pk4[g]], dummy)

        for g in range(3):
            u = lax.bitwise_or(
                lax.bitwise_or(lax.shift_left(b1[g], 21), lax.shift_left(b2[g], 14)),
                lax.bitwise_or(lax.shift_left(b3[g], 7), b4[g]))
            vstage[pl.ds(g * L, L)] = _from_sortable(u)
        vstage[pl.ds(3 * L, L)] = fzeros

        qlo0 = plsc.load_gather(vstage, [lane * 2])
        qhi0 = plsc.load_gather(vstage, [lane * 2 + 1])
        qv0 = qlo0 * (1.0 - hiw0) + qhi0 * hiw0
        qlo1 = plsc.load_gather(vstage, [lane * 2 + 2 * L])
        qhi1 = plsc.load_gather(vstage, [lane * 2 + 2 * L + 1])
        qv1 = qlo1 * (1.0 - hiw1) + qhi1 * hiw1
        qv1 = jnp.where(lane < (NQ - L), qv1, 0.0)
        qrow[i, pl.ds(0, L)] = qv0
        qrow[i, pl.ds(L, L)] = qv1
        return 0

    lax.fori_loop(0, ROWS_PER_TILE, row_body, 0)
    pltpu.sync_copy(qrow, qout_hbm.at[pl.ds(wid * ROWS_PER_TILE, ROWS_PER_TILE)])


def _sc_quantile(x):
    mesh = plsc.VectorSubcoreMesh(core_axis_name="c", subcore_axis_name="s",
                                  num_cores=NC, num_subcores=NS)
    f = pl.kernel(
        _sc_quantile_body,
        out_type=jax.ShapeDtypeStruct((BATCH, QPAD), jnp.float32),
        mesh=mesh,
        compiler_params=pltpu.CompilerParams(needs_layout_passes=False),
        scratch_types=[
            pltpu.VMEM((NPART,), jnp.float32),
            pltpu.VMEM((2048,), jnp.int32),
            pltpu.VMEM((2048,), jnp.int32),
            pltpu.VMEM((NRP * 128,), jnp.int32),
            pltpu.VMEM((NRP * 128,), jnp.int32),
            pltpu.VMEM((NRP * 128,), jnp.int32),
            pltpu.VMEM((64,), jnp.int32),
            pltpu.VMEM((4 * L,), jnp.float32),
            pltpu.VMEM((ROWS_PER_TILE, QPAD), jnp.float32),
            pltpu.VMEM((NRP,), jnp.int32),
            pltpu.VMEM((QPAD,), jnp.float32),
        ],
    )
    hw = np.pad(_HIW, (0, QPAD - NQ)).astype(np.float32)
    return f(x, jnp.asarray(_RANKS), jnp.asarray(hw))


def _mlp_body(q_ref, w1t_ref, b1_ref, w2t_ref, b2_ref, o_ref):
    q = q_ref[...]
    h = jnp.maximum(
        jnp.dot(q, w1t_ref[...], preferred_element_type=jnp.float32) + b1_ref[...],
        0.0)
    o_ref[...] = (jnp.dot(h, w2t_ref[...], preferred_element_type=jnp.float32)
                  + b2_ref[...])


def kernel(x, W1, b1, W2, b2):
    qpad = _sc_quantile(x)
    w1t = jnp.pad(W1.T, ((0, QPAD - NQ), (0, 0)))
    w2t = W2.T
    z = pl.pallas_call(
        _mlp_body,
        out_shape=jax.ShapeDtypeStruct((BATCH, W2.shape[0]), jnp.float32),
    )(qpad, w1t, b1.reshape(1, -1), w2t, b2.reshape(1, -1))
    return z


# speedup vs baseline: 2.3143x; 1.0299x over previous
import numpy as np
import jax
import jax.numpy as jnp
from jax import lax
from jax.experimental import pallas as pl
from jax.experimental.pallas import tpu as pltpu
from jax.experimental.pallas import tpu_sc as plsc

BATCH = 1024
NPART = 16384
NQ = 20
L = 16
NC, NS = 2, 16
NW = NC * NS
ROWS_PER_TILE = BATCH // NW
NVEC = NPART // L
NRP = 48
DUMMY = 47
QPAD = 32
UNROLL = 8
SIGN = np.int32(-2147483648)
MASK31 = np.int32(2147483647)


def _rank_consts():
    qs = np.array([
        1028443341, 1036478745, 1041511909, 1044690750, 1047869591,
        1049812216, 1051401637, 1052991057, 1054580478, 1056169898,
        1057361963, 1058156674, 1058951384, 1059746094, 1060540804,
        1061335514, 1062130225, 1062924935, 1063719645, 1064514355,
    ], dtype=np.uint32).view(np.float32)
    pos = (qs * np.float32(NPART - 1)).astype(np.float32)
    lo = np.clip(np.floor(pos), 0, NPART - 1).astype(np.int64)
    hi = np.clip(np.ceil(pos), 0, NPART - 1).astype(np.int64)
    w = (pos - lo.astype(np.float32)).astype(np.float32)
    ranks = np.empty(NRP, dtype=np.int32)
    ranks[0:2 * NQ:2] = lo
    ranks[1:2 * NQ:2] = hi
    ranks[2 * NQ:] = hi[-1]
    return ranks, w


_RANKS, _HIW = _rank_consts()


def _to_sortable(xf):
    bits = lax.bitcast_convert_type(xf, jnp.int32)
    m = lax.shift_right_arithmetic(bits, 31)
    return lax.bitwise_xor(bits, lax.bitwise_or(SIGN, lax.bitwise_and(MASK31, m)))


def _from_sortable(u):
    m = lax.shift_right_arithmetic(u, 31)
    notm = lax.bitwise_xor(m, np.int32(-1))
    bits = lax.bitwise_xor(u, lax.bitwise_or(SIGN, lax.bitwise_and(MASK31, notm)))
    return lax.bitcast_convert_type(bits, jnp.float32)


def _srl(x, n):
    return lax.shift_right_logical(x, np.int32(n))


def _sc_quantile_body(x_hbm, rk_hbm, hw_hbm, qout_hbm, data, hist1, table2,
                      table3, table4, hist2, stage, vstage, qrow, rk_v, hw_v,
                      dsem):
    wid = lax.axis_index("s") * NC + lax.axis_index("c")
    lane = lax.iota(jnp.int32, L)
    zeros = lane * 0
    ones = zeros + 1
    dummy = zeros + DUMMY
    fzeros = zeros.astype(jnp.float32)
    pltpu.sync_copy(rk_hbm, rk_v)
    pltpu.sync_copy(hw_hbm, hw_v)
    kvecs = [rk_v[pl.ds(i * L, L)] for i in range(3)]
    hiw0 = hw_v[pl.ds(0, L)]
    hiw1 = hw_v[pl.ds(L, L)]

    def init1(i, _):
        for j in range(4):
            hist1[pl.ds((i * 4 + j) * L, L)] = zeros
            table2[pl.ds((i * 4 + j) * L, L)] = dummy
        return 0
    lax.fori_loop(0, 2048 // L // 4, init1, 0)

    def init2(i, _):
        for j in range(4):
            hist2[pl.ds((i * 4 + j) * L, L)] = zeros
            table3[pl.ds((i * 4 + j) * L, L)] = dummy
            table4[pl.ds((i * 4 + j) * L, L)] = dummy
        return 0
    lax.fori_loop(0, (NRP * 128) // L // 4, init2, 0)


    def refine_pass(level, shift, table, pkeys, kks, load_u):
        stage[pl.ds(1, L)] = pkeys[0]
        stage[pl.ds(1 + L, L)] = pkeys[1]
        stage[pl.ds(1 + 2 * L, L)] = pkeys[2]
        sh = [stage[pl.ds(i * L, L)] for i in range(3)]
        f0 = jnp.logical_or(pkeys[0] != sh[0], lane == 0)
        f1 = pkeys[1] != sh[1]
        f2 = pkeys[2] != sh[2]
        fi = [f0.astype(jnp.int32), f1.astype(jnp.int32), f2.astype(jnp.int32)]
        n0 = jnp.sum(fi[0])
        n1 = jnp.sum(fi[1])
        slots = [plsc.cumsum(fi[0]) - 1,
                 plsc.cumsum(fi[1]) + (n0 - 1),
                 plsc.cumsum(fi[2]) + (n0 + n1 - 1)]
        plsc.store_scatter(table, [pkeys[0]], slots[0], mask=f0)
        plsc.store_scatter(table, [pkeys[1]], slots[1], mask=f1)
        plsc.store_scatter(table, [pkeys[2]], slots[2], mask=f2)

        def dp(v, _):
            for j in range(UNROLL):
                u = load_u(v * UNROLL + j)
                s = plsc.load_gather(table2, [_srl(u, 21)])
                if level >= 3:
                    k2 = lax.bitwise_and(_srl(u, 14), np.int32(127))
                    s = plsc.load_gather(table3, [s * 128 + k2])
                if level >= 4:
                    k3 = lax.bitwise_and(_srl(u, 7), np.int32(127))
                    s = plsc.load_gather(table4, [s * 128 + k3])
                key = lax.bitwise_and(_srl(u, shift), np.int32(127))
                plsc.addupdate_scatter(hist2, [key * NRP + s], ones)
            return 0
        lax.fori_loop(0, NVEC // UNROLL, dp, 0)

        def sc(b0, carry):
            acc, bins, bef = carry
            for j in range(2):
                base = (b0 * 2 + j) * NRP
                acc2, bins2, bef2 = [], [], []
                for g in range(3):
                    h = plsc.load_gather(hist2, [slots[g] + base])
                    a = acc[g] + h
                    m = a <= kks[g]
                    acc2.append(a)
                    bins2.append(bins[g] + m.astype(jnp.int32))
                    bef2.append(jnp.where(m, a, bef[g]))
                hist2[pl.ds(base, L)] = zeros
                hist2[pl.ds(base + L, L)] = zeros
                hist2[pl.ds(base + 2 * L, L)] = zeros
                acc, bins, bef = tuple(acc2), tuple(bins2), tuple(bef2)
            return (acc, bins, bef)
        z3 = (zeros, zeros, zeros)
        _, bins, bef = lax.fori_loop(0, 64, sc, (z3, z3, z3))
        new_kks = tuple(kks[g] - bef[g] for g in range(3))
        return bins, new_kks, slots

    row0 = wid * ROWS_PER_TILE
    pltpu.make_async_copy(x_hbm.at[row0], data.at[0], dsem).start()

    def row_body(i, _):
        buf = lax.rem(i, 2)
        pltpu.make_async_copy(x_hbm.at[row0 + i], data.at[buf], dsem).wait()

        @pl.when(i + 1 < ROWS_PER_TILE)
        def _prefetch():
            pltpu.make_async_copy(x_hbm.at[row0 + i + 1], data.at[1 - buf],
                                  dsem).start()

        def load_u(v):
            return _to_sortable(data[buf, pl.ds(v * L, L)])

        def p1(v, _):
            for j in range(UNROLL):
                u = load_u(v * UNROLL + j)
                plsc.addupdate_scatter(hist1, [_srl(u, 21)], ones)
            return 0
        lax.fori_loop(0, NVEC // UNROLL, p1, 0)

        def cs(b, carry):
            h = hist1[pl.ds(b * L, L)]
            hist1[pl.ds(b * L, L)] = plsc.cumsum(h) + carry
            return carry + jnp.sum(h)
        lax.fori_loop(0, 2048 // L, cs, np.int32(0))

        def bsearch(kvec):
            b = zeros
            for j in (1024, 512, 256, 128, 64, 32, 16, 8, 4, 2, 1):
                val = plsc.load_gather(hist1, [b + (j - 1)])
                b = jnp.where(val <= kvec, b + j, b)
            return b
        b1 = [bsearch(kvecs[g]) for g in range(3)]
        kks = []
        for g in range(3):
            prev = plsc.load_gather(hist1, [jnp.maximum(b1[g] - 1, 0)])
            bef = jnp.where(b1[g] > 0, prev, 0)
            kks.append(kvecs[g] - bef)
        kks = tuple(kks)

        def clr(b, _):
            for j in range(4):
                hist1[pl.ds((b * 4 + j) * L, L)] = zeros
            return 0
        lax.fori_loop(0, 2048 // L // 4, clr, 0)

        pk2 = tuple(b1)
        b2, kks, s2 = refine_pass(2, 14, table2, pk2, kks, load_u)
        pk3 = tuple(s2[g] * 128 + b2[g] for g in range(3))
        b3, kks, s3 = refine_pass(3, 7, table3, pk3, kks, load_u)
        pk4 = tuple(s3[g] * 128 + b3[g] for g in range(3))
        b4, kks, s4 = refine_pass(4, 0, table4, pk4, kks, load_u)

        for g in range(3):
            plsc.store_scatter(table2, [pk2[g]], dummy)
            plsc.store_scatter(table3, [pk3[g]], dummy)
            plsc.store_scatter(table4, [pk4[g]], dummy)

        for g in range(3):
            u = lax.bitwise_or(
                lax.bitwise_or(lax.shift_left(b1[g], 21), lax.shift_left(b2[g], 14)),
                lax.bitwise_or(lax.shift_left(b3[g], 7), b4[g]))
            vstage[pl.ds(g * L, L)] = _from_sortable(u)
        vstage[pl.ds(3 * L, L)] = fzeros

        qlo0 = plsc.load_gather(vstage, [lane * 2])
        qhi0 = plsc.load_gather(vstage, [lane * 2 + 1])
        qv0 = qlo0 * (1.0 - hiw0) + qhi0 * hiw0
        qlo1 = plsc.load_gather(vstage, [lane * 2 + 2 * L])
        qhi1 = plsc.load_gather(vstage, [lane * 2 + 2 * L + 1])
        qv1 = qlo1 * (1.0 - hiw1) + qhi1 * hiw1
        qv1 = jnp.where(lane < (NQ - L), qv1, 0.0)
        qrow[i, pl.ds(0, L)] = qv0
        qrow[i, pl.ds(L, L)] = qv1
        return 0

    lax.fori_loop(0, ROWS_PER_TILE, row_body, 0)
    pltpu.sync_copy(qrow, qout_hbm.at[pl.ds(wid * ROWS_PER_TILE, ROWS_PER_TILE)])


def _sc_quantile(x):
    mesh = plsc.VectorSubcoreMesh(core_axis_name="c", subcore_axis_name="s",
                                  num_cores=NC, num_subcores=NS)
    f = pl.kernel(
        _sc_quantile_body,
        out_type=jax.ShapeDtypeStruct((BATCH, QPAD), jnp.float32),
        mesh=mesh,
        compiler_params=pltpu.CompilerParams(needs_layout_passes=False),
        scratch_types=[
            pltpu.VMEM((2, NPART), jnp.float32),
            pltpu.VMEM((2048,), jnp.int32),
            pltpu.VMEM((2048,), jnp.int32),
            pltpu.VMEM((NRP * 128,), jnp.int32),
            pltpu.VMEM((NRP * 128,), jnp.int32),
            pltpu.VMEM((NRP * 128,), jnp.int32),
            pltpu.VMEM((64,), jnp.int32),
            pltpu.VMEM((4 * L,), jnp.float32),
            pltpu.VMEM((ROWS_PER_TILE, QPAD), jnp.float32),
            pltpu.VMEM((NRP,), jnp.int32),
            pltpu.VMEM((QPAD,), jnp.float32),
            pltpu.SemaphoreType.DMA,
        ],
    )
    hw = np.pad(_HIW, (0, QPAD - NQ)).astype(np.float32)
    return f(x, jnp.asarray(_RANKS), jnp.asarray(hw))


def _mlp_body(q_ref, w1t_ref, b1_ref, w2t_ref, b2_ref, o_ref):
    q = q_ref[...]
    h = jnp.maximum(
        jnp.dot(q, w1t_ref[...], preferred_element_type=jnp.float32) + b1_ref[...],
        0.0)
    o_ref[...] = (jnp.dot(h, w2t_ref[...], preferred_element_type=jnp.float32)
                  + b2_ref[...])


def kernel(x, W1, b1, W2, b2):
    qpad = _sc_quantile(x)
    w1t = jnp.pad(W1.T, ((0, QPAD - NQ), (0, 0)))
    w2t = W2.T
    z = pl.pallas_call(
        _mlp_body,
        out_shape=jax.ShapeDtypeStruct((BATCH, W2.shape[0]), jnp.float32),
    )(qpad, w1t, b1.reshape(1, -1), w2t, b2.reshape(1, -1))
    return z


# speedup vs baseline: 7.3942x; 3.1950x over previous
import numpy as np
import jax
import jax.numpy as jnp
from jax import lax
from jax.experimental import pallas as pl
from jax.experimental.pallas import tpu as pltpu
from jax.experimental.pallas import tpu_sc as plsc

BATCH = 1024
NPART = 16384
NQ = 20
L = 16
NC, NS = 2, 16
NW = NC * NS
ROWS_PER_TILE = BATCH // NW
NVEC = NPART // L
NRP = 48
DUMMY = 47
QPAD = 32
UNROLL = 8
SIGN = np.int32(-2147483648)
MASK31 = np.int32(2147483647)


def _rank_consts():
    qs = np.array([
        1028443341, 1036478745, 1041511909, 1044690750, 1047869591,
        1049812216, 1051401637, 1052991057, 1054580478, 1056169898,
        1057361963, 1058156674, 1058951384, 1059746094, 1060540804,
        1061335514, 1062130225, 1062924935, 1063719645, 1064514355,
    ], dtype=np.uint32).view(np.float32)
    pos = (qs * np.float32(NPART - 1)).astype(np.float32)
    lo = np.clip(np.floor(pos), 0, NPART - 1).astype(np.int64)
    hi = np.clip(np.ceil(pos), 0, NPART - 1).astype(np.int64)
    w = (pos - lo.astype(np.float32)).astype(np.float32)
    ranks = np.empty(NRP, dtype=np.int32)
    ranks[0:2 * NQ:2] = lo
    ranks[1:2 * NQ:2] = hi
    ranks[2 * NQ:] = hi[-1]
    return ranks, w


_RANKS, _HIW = _rank_consts()


def _to_sortable(xf):
    bits = lax.bitcast_convert_type(xf, jnp.int32)
    m = lax.shift_right_arithmetic(bits, 31)
    return lax.bitwise_xor(bits, lax.bitwise_or(SIGN, lax.bitwise_and(MASK31, m)))


def _from_sortable(u):
    m = lax.shift_right_arithmetic(u, 31)
    notm = lax.bitwise_xor(m, np.int32(-1))
    bits = lax.bitwise_xor(u, lax.bitwise_or(SIGN, lax.bitwise_and(MASK31, notm)))
    return lax.bitcast_convert_type(bits, jnp.float32)


def _srl(x, n):
    return lax.shift_right_logical(x, np.int32(n))


def _sc_quantile_body(x_hbm, rk_hbm, hw_hbm, qout_hbm, data, hist1, table2,
                      table3, table4, hist2, stage, vstage, qrow, rk_v, hw_v,
                      dsem):
    wid = lax.axis_index("s") * NC + lax.axis_index("c")
    lane = lax.iota(jnp.int32, L)
    zeros = lane * 0
    ones = zeros + 1
    dummy = zeros + DUMMY
    fzeros = zeros.astype(jnp.float32)
    pltpu.sync_copy(rk_hbm, rk_v)
    pltpu.sync_copy(hw_hbm, hw_v)
    kvecs = [rk_v[pl.ds(i * L, L)] for i in range(3)]
    hiw0 = hw_v[pl.ds(0, L)]
    hiw1 = hw_v[pl.ds(L, L)]

    def init1(i, _):
        for j in range(4):
            hist1[pl.ds((i * 4 + j) * L, L)] = zeros
            table2[pl.ds((i * 4 + j) * L, L)] = dummy
        return 0
    lax.fori_loop(0, 2048 // L // 4, init1, 0)

    def init2(i, _):
        for j in range(4):
            hist2[pl.ds((i * 4 + j) * L, L)] = zeros
            table3[pl.ds((i * 4 + j) * L, L)] = dummy
            table4[pl.ds((i * 4 + j) * L, L)] = dummy
        return 0
    lax.fori_loop(0, (NRP * 128) // L // 4, init2, 0)


    def refine_pass(level, shift, table, pkeys, kks, load_u):
        stage[pl.ds(1, L)] = pkeys[0]
        stage[pl.ds(1 + L, L)] = pkeys[1]
        stage[pl.ds(1 + 2 * L, L)] = pkeys[2]
        sh = [stage[pl.ds(i * L, L)] for i in range(3)]
        f0 = jnp.logical_or(pkeys[0] != sh[0], lane == 0)
        f1 = pkeys[1] != sh[1]
        f2 = pkeys[2] != sh[2]
        fi = [f0.astype(jnp.int32), f1.astype(jnp.int32), f2.astype(jnp.int32)]
        n0 = jnp.sum(fi[0])
        n1 = jnp.sum(fi[1])
        slots = [plsc.cumsum(fi[0]) - 1,
                 plsc.cumsum(fi[1]) + (n0 - 1),
                 plsc.cumsum(fi[2]) + (n0 + n1 - 1)]
        plsc.store_scatter(table, [pkeys[0]], slots[0], mask=f0)
        plsc.store_scatter(table, [pkeys[1]], slots[1], mask=f1)
        plsc.store_scatter(table, [pkeys[2]], slots[2], mask=f2)

        def dp(v, _):
            us = [load_u(v * UNROLL + j) for j in range(UNROLL)]
            ss = [plsc.load_gather(table2, [_srl(u, 21)]) for u in us]
            if level >= 3:
                ss = [plsc.load_gather(
                    table3,
                    [s * 128 + lax.bitwise_and(_srl(u, 14), np.int32(127))])
                    for s, u in zip(ss, us)]
            if level >= 4:
                ss = [plsc.load_gather(
                    table4,
                    [s * 128 + lax.bitwise_and(_srl(u, 7), np.int32(127))])
                    for s, u in zip(ss, us)]
            idxs = [lax.bitwise_and(_srl(u, shift), np.int32(127)) * NRP + s
                    for s, u in zip(ss, us)]
            for j in range(UNROLL):
                plsc.addupdate_scatter(hist2, [idxs[j]], ones)
            return 0
        lax.fori_loop(0, NVEC // UNROLL, dp, 0)

        SCU = 4
        def sc(b0, carry):
            acc, bins, bef = carry
            hs = []
            for j in range(SCU):
                base = (b0 * SCU + j) * NRP
                hs.append([plsc.load_gather(hist2, [slots[g] + base])
                           for g in range(3)])
            for j in range(SCU):
                acc2, bins2, bef2 = [], [], []
                for g in range(3):
                    a = acc[g] + hs[j][g]
                    m = a <= kks[g]
                    acc2.append(a)
                    bins2.append(bins[g] + m.astype(jnp.int32))
                    bef2.append(jnp.where(m, a, bef[g]))
                acc, bins, bef = tuple(acc2), tuple(bins2), tuple(bef2)
            for j in range(SCU):
                base = (b0 * SCU + j) * NRP
                hist2[pl.ds(base, L)] = zeros
                hist2[pl.ds(base + L, L)] = zeros
                hist2[pl.ds(base + 2 * L, L)] = zeros
            return (acc, bins, bef)
        z3 = (zeros, zeros, zeros)
        _, bins, bef = lax.fori_loop(0, 128 // SCU, sc, (z3, z3, z3))
        new_kks = tuple(kks[g] - bef[g] for g in range(3))
        return bins, new_kks, slots

    row0 = wid * ROWS_PER_TILE
    pltpu.make_async_copy(x_hbm.at[row0], data.at[0], dsem).start()

    def row_body(i, _):
        buf = lax.rem(i, 2)
        pltpu.make_async_copy(x_hbm.at[row0 + i], data.at[buf], dsem).wait()

        @pl.when(i + 1 < ROWS_PER_TILE)
        def _prefetch():
            pltpu.make_async_copy(x_hbm.at[row0 + i + 1], data.at[1 - buf],
                                  dsem).start()

        def load_u(v):
            return _to_sortable(data[buf, pl.ds(v * L, L)])

        def p1(v, _):
            keys = [_srl(load_u(v * UNROLL + j), 21) for j in range(UNROLL)]
            for j in range(UNROLL):
                plsc.addupdate_scatter(hist1, [keys[j]], ones)
            return 0
        lax.fori_loop(0, NVEC // UNROLL, p1, 0)

        def cs(b, carry):
            h = hist1[pl.ds(b * L, L)]
            hist1[pl.ds(b * L, L)] = plsc.cumsum(h) + carry
            return carry + jnp.sum(h)
        lax.fori_loop(0, 2048 // L, cs, np.int32(0))

        def bsearch(kvec):
            b = zeros
            for j in (1024, 512, 256, 128, 64, 32, 16, 8, 4, 2, 1):
                val = plsc.load_gather(hist1, [b + (j - 1)])
                b = jnp.where(val <= kvec, b + j, b)
            return b
        b1 = [bsearch(kvecs[g]) for g in range(3)]
        kks = []
        for g in range(3):
            prev = plsc.load_gather(hist1, [jnp.maximum(b1[g] - 1, 0)])
            bef = jnp.where(b1[g] > 0, prev, 0)
            kks.append(kvecs[g] - bef)
        kks = tuple(kks)

        def clr(b, _):
            for j in range(4):
                hist1[pl.ds((b * 4 + j) * L, L)] = zeros
            return 0
        lax.fori_loop(0, 2048 // L // 4, clr, 0)

        pk2 = tuple(b1)
        b2, kks, s2 = refine_pass(2, 14, table2, pk2, kks, load_u)
        pk3 = tuple(s2[g] * 128 + b2[g] for g in range(3))
        b3, kks, s3 = refine_pass(3, 7, table3, pk3, kks, load_u)
        pk4 = tuple(s3[g] * 128 + b3[g] for g in range(3))
        b4, kks, s4 = refine_pass(4, 0, table4, pk4, kks, load_u)

        for g in range(3):
            plsc.store_scatter(table2, [pk2[g]], dummy)
            plsc.store_scatter(table3, [pk3[g]], dummy)
            plsc.store_scatter(table4, [pk4[g]], dummy)

        for g in range(3):
            u = lax.bitwise_or(
                lax.bitwise_or(lax.shift_left(b1[g], 21), lax.shift_left(b2[g], 14)),
                lax.bitwise_or(lax.shift_left(b3[g], 7), b4[g]))
            vstage[pl.ds(g * L, L)] = _from_sortable(u)
        vstage[pl.ds(3 * L, L)] = fzeros

        qlo0 = plsc.load_gather(vstage, [lane * 2])
        qhi0 = plsc.load_gather(vstage, [lane * 2 + 1])
        qv0 = qlo0 * (1.0 - hiw0) + qhi0 * hiw0
        qlo1 = plsc.load_gather(vstage, [lane * 2 + 2 * L])
        qhi1 = plsc.load_gather(vstage, [lane * 2 + 2 * L + 1])
        qv1 = qlo1 * (1.0 - hiw1) + qhi1 * hiw1
        qv1 = jnp.where(lane < (NQ - L), qv1, 0.0)
        qrow[i, pl.ds(0, L)] = qv0
        qrow[i, pl.ds(L, L)] = qv1
        return 0

    lax.fori_loop(0, ROWS_PER_TILE, row_body, 0)
    pltpu.sync_copy(qrow, qout_hbm.at[pl.ds(wid * ROWS_PER_TILE, ROWS_PER_TILE)])


def _sc_quantile(x):
    mesh = plsc.VectorSubcoreMesh(core_axis_name="c", subcore_axis_name="s",
                                  num_cores=NC, num_subcores=NS)
    f = pl.kernel(
        _sc_quantile_body,
        out_type=jax.ShapeDtypeStruct((BATCH, QPAD), jnp.float32),
        mesh=mesh,
        compiler_params=pltpu.CompilerParams(needs_layout_passes=False),
        scratch_types=[
            pltpu.VMEM((2, NPART), jnp.float32),
            pltpu.VMEM((2048,), jnp.int32),
            pltpu.VMEM((2048,), jnp.int32),
            pltpu.VMEM((NRP * 128,), jnp.int32),
            pltpu.VMEM((NRP * 128,), jnp.int32),
            pltpu.VMEM((NRP * 128,), jnp.int32),
            pltpu.VMEM((64,), jnp.int32),
            pltpu.VMEM((4 * L,), jnp.float32),
            pltpu.VMEM((ROWS_PER_TILE, QPAD), jnp.float32),
            pltpu.VMEM((NRP,), jnp.int32),
            pltpu.VMEM((QPAD,), jnp.float32),
            pltpu.SemaphoreType.DMA,
        ],
    )
    hw = np.pad(_HIW, (0, QPAD - NQ)).astype(np.float32)
    return f(x, jnp.asarray(_RANKS), jnp.asarray(hw))


def _mlp_body(q_ref, w1t_ref, b1_ref, w2t_ref, b2_ref, o_ref):
    q = q_ref[...]
    h = jnp.maximum(
        jnp.dot(q, w1t_ref[...], preferred_element_type=jnp.float32) + b1_ref[...],
        0.0)
    o_ref[...] = (jnp.dot(h, w2t_ref[...], preferred_element_type=jnp.float32)
                  + b2_ref[...])


def kernel(x, W1, b1, W2, b2):
    qpad = _sc_quantile(x)
    w1t = jnp.pad(W1.T, ((0, QPAD - NQ), (0, 0)))
    w2t = W2.T
    z = pl.pallas_call(
        _mlp_body,
        out_shape=jax.ShapeDtypeStruct((BATCH, W2.shape[0]), jnp.float32),
    )(qpad, w1t, b1.reshape(1, -1), w2t, b2.reshape(1, -1))
    return z
